# baseline (speedup 1.0000x reference)
import functools

import jax
import jax.numpy as jnp
from jax import lax
from jax.experimental import pallas as pl
from jax.experimental.pallas import tpu as pltpu
from jax.experimental.pallas import tpu_sc as plsc

NUM_CLASSES = 1000
NUM_ROWS = 16384

_NC = 2
_NS = 16
_NW = _NC * _NS
_ROWS_PER_W = NUM_ROWS // _NW
_R = 64
_NB = _ROWS_PER_W // _R
_NBUF = 2
_L = 16


def _one_hot_body(x1_hbm, zeros_hbm, out_hbm,
                  buf0, buf1, idx0, idx1, sem0, sem1, zsem):
    bufs = (buf0, buf1)
    idxs = (idx0, idx1)
    sems = (sem0, sem1)

    wid = lax.axis_index("s") * _NC + lax.axis_index("c")
    base = wid * _ROWS_PER_W

    zcp0 = pltpu.async_copy(zeros_hbm, buf0, zsem)
    zcp1 = pltpu.async_copy(zeros_hbm, buf1, zsem)

    row_iota = lax.iota(jnp.int32, _L)
    ones_v = jnp.full((_L,), 1.0, jnp.float32)
    zeros_v = jnp.zeros((_L,), jnp.float32)

    def scatter(buf, idx_ref, vals):
        for j in range(_R // _L):
            rows = row_iota + (j * _L)
            cols = idx_ref[pl.ds(j * _L, _L)]
            plsc.store_scatter(buf, [rows, cols], vals)

    zcp0.wait()
    zcp1.wait()

    out_cps = [None, None]
    for g in range(_NB):
        b = g % _NBUF
        if out_cps[b] is not None:
            out_cps[b].wait()
            scatter(bufs[b], idxs[b], zeros_v)
        pltpu.sync_copy(x1_hbm.at[pl.ds(base + g * _R, _R)], idxs[b])
        scatter(bufs[b], idxs[b], ones_v)
        out_cps[b] = pltpu.async_copy(
            bufs[b], out_hbm.at[pl.ds(base + g * _R, _R)], sems[b])
    for b in range(_NBUF):
        out_cps[b].wait()


@jax.jit
def kernel(x1):
    x1 = x1.astype(jnp.int32)
    zeros = jnp.zeros((_R, NUM_CLASSES), jnp.float32)
    mesh = plsc.VectorSubcoreMesh(core_axis_name="c", subcore_axis_name="s")
    f = pl.kernel(
        _one_hot_body,
        out_type=jax.ShapeDtypeStruct((NUM_ROWS, NUM_CLASSES), jnp.float32),
        mesh=mesh,
        scratch_types=[
            pltpu.VMEM((_R, NUM_CLASSES), jnp.float32),
            pltpu.VMEM((_R, NUM_CLASSES), jnp.float32),
            pltpu.VMEM((_R,), jnp.int32),
            pltpu.VMEM((_R,), jnp.int32),
            pltpu.SemaphoreType.DMA,
            pltpu.SemaphoreType.DMA,
            pltpu.SemaphoreType.DMA,
        ],
        compiler_params=pltpu.CompilerParams(
            use_tc_tiling_on_sc=False, needs_layout_passes=False),
    )
    return f(x1, zeros)


# speedup vs baseline: 7.3492x; 7.3492x over previous
import jax
import jax.numpy as jnp
from jax import lax
from jax.experimental import pallas as pl
from jax.experimental.pallas import tpu as pltpu

NUM_CLASSES = 1000
NUM_ROWS = 16384

_BLKC = 200
_GRID = NUM_CLASSES // _BLKC


def _body(x1_ref, out_ref):
    ids = x1_ref[0, 0]
    base = pl.program_id(0) * _BLKC
    cls = lax.broadcasted_iota(jnp.int32, (_BLKC, NUM_ROWS), 0) + base
    out_ref[...] = (cls == ids[None, :]).astype(jnp.float32)


@jax.jit
def kernel(x1):
    x1 = x1.astype(jnp.int32)
    x1r = x1.reshape(1, 1, NUM_ROWS)
    out_t = pl.pallas_call(
        _body,
        grid=(_GRID,),
        in_specs=[pl.BlockSpec((1, 1, NUM_ROWS), lambda i: (0, 0, 0))],
        out_specs=pl.BlockSpec((_BLKC, NUM_ROWS), lambda i: (i, 0)),
        out_shape=jax.ShapeDtypeStruct((NUM_CLASSES, NUM_ROWS), jnp.float32),
    )(x1r)
    return out_t.T
